# baseline (speedup 1.0000x reference)
import functools

import jax
import jax.numpy as jnp
from jax import lax
from jax.experimental import pallas as pl
from jax.experimental.pallas import tpu as pltpu
from jax.experimental.pallas import tpu_sc as plsc

B = 16384
U = 1000000
DG = 16
DM = 64

_info = plsc.get_sparse_core_info()
NC = _info.num_cores
NS = _info.num_subcores
NW = NC * NS
RPW = B // NW
CH = 128
NCH = RPW // CH

UB = 8192
NUB = -(-U // UB)


def _tr_body(tu_ref, ti_ref, gu_ref, gi_ref, otu_ref, oti_ref, ogu_ref,
             ogi_ref):
    otu_ref[...] = tu_ref[...].astype(jnp.bfloat16).T
    oti_ref[...] = ti_ref[...].astype(jnp.bfloat16).T
    ogu_ref[...] = gu_ref[...].T
    ogi_ref[...] = gi_ref[...].T


def _tc_transpose(tuT, tiT, guT, giT):
    col = lambda i: (0, i)
    row = lambda i: (i, 0)
    return pl.pallas_call(
        _tr_body,
        grid=(NUB,),
        in_specs=[
            pl.BlockSpec((DM, UB), col),
            pl.BlockSpec((DM, UB), col),
            pl.BlockSpec((DG, UB), col),
            pl.BlockSpec((DG, UB), col),
        ],
        out_specs=[
            pl.BlockSpec((UB, DM), row),
            pl.BlockSpec((UB, DM), row),
            pl.BlockSpec((UB, DG), row),
            pl.BlockSpec((UB, DG), row),
        ],
        out_shape=[
            jax.ShapeDtypeStruct((U, DM), jnp.bfloat16),
            jax.ShapeDtypeStruct((U, DM), jnp.bfloat16),
            jax.ShapeDtypeStruct((U, DG), jnp.float32),
            jax.ShapeDtypeStruct((U, DG), jnp.float32),
        ],
    )(tuT, tiT, guT, giT)


@functools.partial(
    pl.kernel,
    out_type=(
        jax.ShapeDtypeStruct((B, DM), jnp.bfloat16),
        jax.ShapeDtypeStruct((B, DM), jnp.bfloat16),
        jax.ShapeDtypeStruct((B, DG), jnp.float32),
        jax.ShapeDtypeStruct((B, DG), jnp.float32),
    ),
    mesh=plsc.VectorSubcoreMesh(core_axis_name="c", subcore_axis_name="s"),
    compiler_params=pltpu.CompilerParams(use_tc_tiling_on_sc=False),
    scratch_types=[
        pltpu.VMEM((NCH, CH), jnp.int32),
        pltpu.VMEM((NCH, CH), jnp.int32),
        pltpu.VMEM((RPW, DM), jnp.bfloat16),
        pltpu.VMEM((RPW, DM), jnp.bfloat16),
        pltpu.VMEM((RPW, DG), jnp.float32),
        pltpu.VMEM((RPW, DG), jnp.float32),
        pltpu.SemaphoreType.DMA,
        pltpu.SemaphoreType.DMA,
        pltpu.SemaphoreType.DMA,
        pltpu.SemaphoreType.DMA,
    ],
)
def _sc_gather(user_hbm, item_hbm, eum_hbm, eim_hbm, eug_hbm, eig_hbm,
               um_out, im_out, ug_out, ig_out,
               uidx, iidx, um_v, im_v, ug_v, ig_v, s0, s1, s2, s3):
    wid = lax.axis_index("s") * NC + lax.axis_index("c")
    base = wid * RPW
    pltpu.sync_copy(user_hbm.at[wid], uidx)
    pltpu.sync_copy(item_hbm.at[wid], iidx)
    cps = []
    for j in range(NCH):
        sl = pl.ds(j * CH, CH)
        cps.append(pltpu.async_copy(eum_hbm.at[uidx.at[j]], um_v.at[sl], s0))
        cps.append(pltpu.async_copy(eim_hbm.at[iidx.at[j]], im_v.at[sl], s1))
        cps.append(pltpu.async_copy(eug_hbm.at[uidx.at[j]], ug_v.at[sl], s2))
        cps.append(pltpu.async_copy(eig_hbm.at[iidx.at[j]], ig_v.at[sl], s3))
    for cp in cps:
        cp.wait()
    pltpu.sync_copy(um_v, um_out.at[pl.ds(base, RPW)])
    pltpu.sync_copy(im_v, im_out.at[pl.ds(base, RPW)])
    pltpu.sync_copy(ug_v, ug_out.at[pl.ds(base, RPW)])
    pltpu.sync_copy(ig_v, ig_out.at[pl.ds(base, RPW)])


def _dense_body(um_ref, im_ref, ug_ref, ig_ref, w1u_ref, w1i_ref, b1_ref,
                w2_ref, b2_ref, w3_ref, b3_ref, wpg_ref, wph_ref, bp_ref,
                o_ref):
    f32 = jnp.float32
    h = (jnp.dot(um_ref[...], w1u_ref[...], preferred_element_type=f32)
         + jnp.dot(im_ref[...], w1i_ref[...], preferred_element_type=f32)
         + b1_ref[...])
    h = jnp.maximum(h, 0.0)
    h = jnp.maximum(h @ w2_ref[...] + b2_ref[...], 0.0)
    h = jnp.maximum(h @ w3_ref[...] + b3_ref[...], 0.0)
    gmf = ug_ref[...] * ig_ref[...]
    z = gmf @ wpg_ref[...] + h @ wph_ref[...] + bp_ref[...]
    o_ref[...] = 1.0 / (1.0 + jnp.exp(-z))


def _tc_dense(um, im, ug, ig, w1u, w1i, b1, w2, b2, w3, b3, wpg, wph, bp):
    BLK = 2048
    row = lambda i: (i, 0)
    rep = lambda i: (0, 0)
    return pl.pallas_call(
        _dense_body,
        grid=(B // BLK,),
        in_specs=[
            pl.BlockSpec((BLK, DM), row),
            pl.BlockSpec((BLK, DM), row),
            pl.BlockSpec((BLK, DG), row),
            pl.BlockSpec((BLK, DG), row),
            pl.BlockSpec((DM, DM), rep),
            pl.BlockSpec((DM, DM), rep),
            pl.BlockSpec((1, DM), rep),
            pl.BlockSpec((DM, DM // 2), rep),
            pl.BlockSpec((1, DM // 2), rep),
            pl.BlockSpec((DM // 2, DG), rep),
            pl.BlockSpec((1, DG), rep),
            pl.BlockSpec((DG, 1), rep),
            pl.BlockSpec((DG, 1), rep),
            pl.BlockSpec((1, 1), rep),
        ],
        out_specs=pl.BlockSpec((BLK, 1), row),
        out_shape=jax.ShapeDtypeStruct((B, 1), jnp.float32),
    )(um, im, ug, ig, w1u, w1i, b1, w2, b2, w3, b3, wpg, wph, bp)


def kernel(user, item, embed_user_GMF, embed_item_GMF, embed_user_MLP,
           embed_item_MLP, W1, b1, W2, b2, W3, b3, Wp, bp):
    eum, eim, eug, eig = _tc_transpose(
        embed_user_MLP.T, embed_item_MLP.T,
        embed_user_GMF.T, embed_item_GMF.T)
    user3 = user.astype(jnp.int32).reshape(NW, NCH, CH)
    item3 = item.astype(jnp.int32).reshape(NW, NCH, CH)
    um, im, ug, ig = _sc_gather(user3, item3, eum, eim, eug, eig)
    out = _tc_dense(
        um, im, ug, ig,
        W1[:DM].astype(jnp.bfloat16), W1[DM:].astype(jnp.bfloat16),
        b1.reshape(1, DM),
        W2, b2.reshape(1, DM // 2),
        W3, b3.reshape(1, DG),
        Wp[:DG], Wp[DG:], bp.reshape(1, 1),
    )
    return out.reshape(-1)


# speedup vs baseline: 3.6400x; 3.6400x over previous
import functools

import jax
import jax.numpy as jnp
from jax import lax
from jax.experimental import pallas as pl
from jax.experimental.pallas import tpu as pltpu
from jax.experimental.pallas import tpu_sc as plsc

B = 16384
U = 1000000
DG = 16
DM = 64

_info = plsc.get_sparse_core_info()
NC = _info.num_cores
NS = _info.num_subcores
NW = NC * NS
RPW = B // NW
CH = 128
NCH = RPW // CH

UB = 8192
NUB = -(-U // UB)


def _eye(n):
    return (jax.lax.broadcasted_iota(jnp.int32, (n, n), 0)
            == jax.lax.broadcasted_iota(jnp.int32, (n, n), 1)).astype(jnp.float32)


def _dot_t(x, ident):
    return jax.lax.dot_general(x, ident, (((0,), (0,)), ((), ())),
                               preferred_element_type=jnp.float32)


def _pk_body(tu_ref, ti_ref, gu_ref, gi_ref, ou_ref, oi_ref):
    i_m = _eye(DM)
    i_g = _eye(DG)
    pad = jnp.zeros((tu_ref.shape[1], 128 - DM - DG), jnp.float32)
    ou_ref[...] = jnp.concatenate(
        [_dot_t(tu_ref[...], i_m),
         gu_ref[...].astype(jnp.bfloat16).T.astype(jnp.float32), pad], axis=1)
    oi_ref[...] = jnp.concatenate(
        [_dot_t(ti_ref[...], i_m),
         gi_ref[...].astype(jnp.bfloat16).T.astype(jnp.float32), pad], axis=1)


def _tc_pack(tuT, tiT, guT, giT):
    col = lambda i: (0, i)
    row = lambda i: (i, 0)
    return pl.pallas_call(
        _pk_body,
        grid=(NUB,),
        compiler_params=pltpu.CompilerParams(
            fuse_transposed_lhs_in_matmul=True),
        in_specs=[
            pl.BlockSpec((DM, UB), col),
            pl.BlockSpec((DM, UB), col),
            pl.BlockSpec((DG, UB), col),
            pl.BlockSpec((DG, UB), col),
        ],
        out_specs=[
            pl.BlockSpec((UB, 128), row),
            pl.BlockSpec((UB, 128), row),
        ],
        out_shape=[
            jax.ShapeDtypeStruct((U, 128), jnp.float32),
            jax.ShapeDtypeStruct((U, 128), jnp.float32),
        ],
    )(tuT, tiT, guT, giT)


@functools.partial(
    pl.kernel,
    out_type=(
        jax.ShapeDtypeStruct((B, 128), jnp.float32),
        jax.ShapeDtypeStruct((B, 128), jnp.float32),
    ),
    mesh=plsc.VectorSubcoreMesh(core_axis_name="c", subcore_axis_name="s"),
    scratch_types=[
        pltpu.VMEM((NCH, CH), jnp.int32),
        pltpu.VMEM((NCH, CH), jnp.int32),
        pltpu.VMEM((CH, 128), jnp.float32),
        pltpu.VMEM((CH, 128), jnp.float32),
        pltpu.SemaphoreType.DMA,
        pltpu.SemaphoreType.DMA,
    ],
)
def _sc_gather(uidx_hbm, iidx_hbm, tu_hbm, ti_hbm,
               um_out, im_out,
               uidx, iidx, b0, b1, s0, s1):
    wid = lax.axis_index("s") * NC + lax.axis_index("c")
    base = wid * RPW
    pltpu.sync_copy(uidx_hbm.at[wid], uidx)
    pltpu.sync_copy(iidx_hbm.at[wid], iidx)
    for j in range(NCH):
        cps = [
            pltpu.async_copy(tu_hbm.at[uidx.at[j]], b0, s0),
            pltpu.async_copy(ti_hbm.at[iidx.at[j]], b1, s1),
        ]
        for cp in cps:
            cp.wait()
        sl = pl.ds(base + j * CH, CH)
        pltpu.sync_copy(b0, um_out.at[sl])
        pltpu.sync_copy(b1, im_out.at[sl])


def _dense_body(um_ref, im_ref, w1u_ref, w1i_ref, b1_ref, w2_ref, b2_ref,
                w3_ref, b3_ref, wpg_ref, wph_ref, bp_ref, o_ref):
    h = (um_ref[...][:, :DM] @ w1u_ref[...]
         + im_ref[...][:, :DM] @ w1i_ref[...] + b1_ref[...])
    h = jnp.maximum(h, 0.0)
    h = jnp.maximum(h @ w2_ref[...] + b2_ref[...], 0.0)
    h = jnp.maximum(h @ w3_ref[...] + b3_ref[...], 0.0)
    gmf = um_ref[...][:, DM:DM + DG] * im_ref[...][:, DM:DM + DG]
    z = gmf @ wpg_ref[...] + h @ wph_ref[...] + bp_ref[...]
    o_ref[...] = 1.0 / (1.0 + jnp.exp(-z))


def _tc_dense(um, im, w1u, w1i, b1, w2, b2, w3, b3, wpg, wph, bp):
    BLK = 2048
    row = lambda i: (i, 0)
    rep = lambda i: (0, 0)
    return pl.pallas_call(
        _dense_body,
        grid=(B // BLK,),
        in_specs=[
            pl.BlockSpec((BLK, 128), row),
            pl.BlockSpec((BLK, 128), row),
            pl.BlockSpec((DM, DM), rep),
            pl.BlockSpec((DM, DM), rep),
            pl.BlockSpec((1, DM), rep),
            pl.BlockSpec((DM, DM // 2), rep),
            pl.BlockSpec((1, DM // 2), rep),
            pl.BlockSpec((DM // 2, DG), rep),
            pl.BlockSpec((1, DG), rep),
            pl.BlockSpec((DG, 1), rep),
            pl.BlockSpec((DG, 1), rep),
            pl.BlockSpec((1, 1), rep),
        ],
        out_specs=pl.BlockSpec((BLK, 1), row),
        out_shape=jax.ShapeDtypeStruct((B, 1), jnp.float32),
    )(um, im, w1u, w1i, b1, w2, b2, w3, b3, wpg, wph, bp)


def kernel(user, item, embed_user_GMF, embed_item_GMF, embed_user_MLP,
           embed_item_MLP, W1, b1, W2, b2, W3, b3, Wp, bp):
    t_user, t_item = _tc_pack(
        embed_user_MLP.T, embed_item_MLP.T,
        embed_user_GMF.T, embed_item_GMF.T)
    um_idx = user.astype(jnp.int32).reshape(NW, NCH, CH)
    im_idx = item.astype(jnp.int32).reshape(NW, NCH, CH)
    um_g, im_g = _sc_gather(um_idx, im_idx, t_user, t_item)
    out = _tc_dense(
        um_g, im_g,
        W1[:DM], W1[DM:], b1.reshape(1, DM),
        W2, b2.reshape(1, DM // 2),
        W3, b3.reshape(1, DG),
        Wp[:DG], Wp[DG:], bp.reshape(1, 1),
    )
    return out.reshape(-1)


# speedup vs baseline: 4.5307x; 1.2447x over previous
import functools

import jax
import jax.numpy as jnp
from jax import lax
from jax.experimental import pallas as pl
from jax.experimental.pallas import tpu as pltpu
from jax.experimental.pallas import tpu_sc as plsc

B = 16384
U = 1000000
DG = 16
DM = 64

_info = plsc.get_sparse_core_info()
NC = _info.num_cores
NS = _info.num_subcores
NW = NC * NS
RPW = B // NW
CH = 128
NCH = RPW // CH

UB = 8192
NUB = -(-U // UB)


def _pair_pack(x_ref):
    xb = x_ref[...].astype(jnp.bfloat16).T
    return pltpu.bitcast(xb, jnp.float32)


def _pk_body(tu_ref, ti_ref, gu_ref, gi_ref, ou_ref, oi_ref):
    pad = jnp.zeros((UB // 2, 128 - DM - DG), jnp.float32)
    ou_ref[...] = jnp.concatenate(
        [_pair_pack(tu_ref), _pair_pack(gu_ref), pad], axis=1)
    oi_ref[...] = jnp.concatenate(
        [_pair_pack(ti_ref), _pair_pack(gi_ref), pad], axis=1)


def _tc_pack(tuT, tiT, guT, giT):
    col = lambda i: (0, i)
    row = lambda i: (i, 0)
    return pl.pallas_call(
        _pk_body,
        grid=(NUB,),
        in_specs=[
            pl.BlockSpec((DM, UB), col),
            pl.BlockSpec((DM, UB), col),
            pl.BlockSpec((DG, UB), col),
            pl.BlockSpec((DG, UB), col),
        ],
        out_specs=[
            pl.BlockSpec((UB // 2, 128), row),
            pl.BlockSpec((UB // 2, 128), row),
        ],
        out_shape=[
            jax.ShapeDtypeStruct((U // 2, 128), jnp.float32),
            jax.ShapeDtypeStruct((U // 2, 128), jnp.float32),
        ],
    )(tuT, tiT, guT, giT)


@functools.partial(
    pl.kernel,
    out_type=(
        jax.ShapeDtypeStruct((B, 128), jnp.float32),
        jax.ShapeDtypeStruct((B, 128), jnp.float32),
    ),
    mesh=plsc.VectorSubcoreMesh(core_axis_name="c", subcore_axis_name="s"),
    scratch_types=[
        pltpu.VMEM((NCH, CH), jnp.int32),
        pltpu.VMEM((NCH, CH), jnp.int32),
        pltpu.VMEM((CH, 128), jnp.float32),
        pltpu.VMEM((CH, 128), jnp.float32),
        pltpu.SemaphoreType.DMA,
        pltpu.SemaphoreType.DMA,
    ],
)
def _sc_gather(uidx_hbm, iidx_hbm, tu_hbm, ti_hbm,
               um_out, im_out,
               uidx, iidx, b0, b1, s0, s1):
    wid = lax.axis_index("s") * NC + lax.axis_index("c")
    base = wid * RPW
    pltpu.sync_copy(uidx_hbm.at[wid], uidx)
    pltpu.sync_copy(iidx_hbm.at[wid], iidx)
    for j in range(NCH):
        cps = [
            pltpu.async_copy(tu_hbm.at[uidx.at[j]], b0, s0),
            pltpu.async_copy(ti_hbm.at[iidx.at[j]], b1, s1),
        ]
        for cp in cps:
            cp.wait()
        sl = pl.ds(base + j * CH, CH)
        pltpu.sync_copy(b0, um_out.at[sl])
        pltpu.sync_copy(b1, im_out.at[sl])


def _unpack_half(x, sel_hi):
    xi = jax.lax.bitcast_convert_type(x, jnp.int32)
    hi = jax.lax.bitcast_convert_type((xi >> 16) << 16, jnp.float32)
    lo = jax.lax.bitcast_convert_type(xi << 16, jnp.float32)
    return jnp.where(sel_hi, hi, lo)


def _dense_body(um_ref, im_ref, u1_ref, i1_ref, w1u_ref, w1i_ref, b1_ref,
                w2_ref, b2_ref, w3_ref, b3_ref, wpg_ref, wph_ref, bp_ref,
                o_ref):
    usel = u1_ref[...] > 0.5
    isel = i1_ref[...] > 0.5
    u_mlp = _unpack_half(um_ref[...][:, :DM], usel)
    u_gmf = _unpack_half(um_ref[...][:, DM:DM + DG], usel)
    i_mlp = _unpack_half(im_ref[...][:, :DM], isel)
    i_gmf = _unpack_half(im_ref[...][:, DM:DM + DG], isel)
    h = u_mlp @ w1u_ref[...] + i_mlp @ w1i_ref[...] + b1_ref[...]
    h = jnp.maximum(h, 0.0)
    h = jnp.maximum(h @ w2_ref[...] + b2_ref[...], 0.0)
    h = jnp.maximum(h @ w3_ref[...] + b3_ref[...], 0.0)
    gmf = u_gmf * i_gmf
    z = gmf @ wpg_ref[...] + h @ wph_ref[...] + bp_ref[...]
    o_ref[...] = 1.0 / (1.0 + jnp.exp(-z))


def _tc_dense(um, im, u1, i1, w1u, w1i, b1, w2, b2, w3, b3, wpg, wph, bp):
    BLK = 2048
    row = lambda i: (i, 0)
    rep = lambda i: (0, 0)
    return pl.pallas_call(
        _dense_body,
        grid=(B // BLK,),
        in_specs=[
            pl.BlockSpec((BLK, 128), row),
            pl.BlockSpec((BLK, 128), row),
            pl.BlockSpec((BLK, 1), row),
            pl.BlockSpec((BLK, 1), row),
            pl.BlockSpec((DM, DM), rep),
            pl.BlockSpec((DM, DM), rep),
            pl.BlockSpec((1, DM), rep),
            pl.BlockSpec((DM, DM // 2), rep),
            pl.BlockSpec((1, DM // 2), rep),
            pl.BlockSpec((DM // 2, DG), rep),
            pl.BlockSpec((1, DG), rep),
            pl.BlockSpec((DG, 1), rep),
            pl.BlockSpec((DG, 1), rep),
            pl.BlockSpec((1, 1), rep),
        ],
        out_specs=pl.BlockSpec((BLK, 1), row),
        out_shape=jax.ShapeDtypeStruct((B, 1), jnp.float32),
    )(um, im, u1, i1, w1u, w1i, b1, w2, b2, w3, b3, wpg, wph, bp)


def kernel(user, item, embed_user_GMF, embed_item_GMF, embed_user_MLP,
           embed_item_MLP, W1, b1, W2, b2, W3, b3, Wp, bp):
    t_user, t_item = _tc_pack(
        embed_user_MLP.T, embed_item_MLP.T,
        embed_user_GMF.T, embed_item_GMF.T)
    ui = user.astype(jnp.int32)
    ii = item.astype(jnp.int32)
    um_idx = (ui >> 1).reshape(NW, NCH, CH)
    im_idx = (ii >> 1).reshape(NW, NCH, CH)
    um_g, im_g = _sc_gather(um_idx, im_idx, t_user, t_item)
    u1 = (ui & 1).astype(jnp.float32).reshape(B, 1)
    i1 = (ii & 1).astype(jnp.float32).reshape(B, 1)
    out = _tc_dense(
        um_g, im_g, u1, i1,
        W1[:DM], W1[DM:], b1.reshape(1, DM),
        W2, b2.reshape(1, DM // 2),
        W3, b3.reshape(1, DG),
        Wp[:DG], Wp[DG:], bp.reshape(1, 1),
    )
    return out.reshape(-1)
